# baseline (speedup 1.0000x reference)
import functools

import jax
import jax.numpy as jnp
from jax import lax
from jax.experimental import pallas as pl
from jax.experimental.pallas import tpu as pltpu
from jax.experimental.pallas import tpu_sc as plsc

BATCH = 16384
EMB = 64
HALF = EMB // 2
NC = 2
NS = 16
NW = NC * NS
BPW = BATCH // NW
WAVE = 256
NWAVE = BPW // WAVE
CHUNK = 128
LANES = 16

_mesh = plsc.VectorSubcoreMesh(core_axis_name="c", subcore_axis_name="s")
_params = pltpu.CompilerParams(needs_layout_passes=False)


@functools.partial(
    pl.kernel,
    out_type=pltpu.HBM((NW, BPW, HALF), jnp.int32),
    mesh=_mesh,
    compiler_params=_params,
    scratch_types=[
        pltpu.VMEM((BPW,), jnp.int32),
        pltpu.VMEM((BPW,), jnp.int32),
        pltpu.VMEM((WAVE, 128), jnp.float32),
        pltpu.VMEM((BPW, HALF), jnp.int32),
        pltpu.SemaphoreType.DMA,
    ],
)
def _mf_gather_u(u_id_hbm, utab_hbm, stage_hbm,
                 uid_v, ukey_v, udat_v, ustg_v, sem):
    wid = lax.axis_index("s") * NC + lax.axis_index("c")
    pltpu.sync_copy(u_id_hbm.at[wid], uid_v)
    for o in range(BPW // LANES):
        sl = pl.ds(o * LANES, LANES)
        ukey_v[sl] = lax.shift_right_logical(uid_v[sl], 1)

    lane = lax.iota(jnp.int32, LANES)

    def wave_body(wv, carry):
        base = wv * WAVE
        copies = []
        for c in range(WAVE // CHUNK):
            sl = pl.ds(base + c * CHUNK, CHUNK)
            copies.append(
                pltpu.async_copy(utab_hbm.at[ukey_v.at[sl]],
                                 udat_v.at[pl.ds(c * CHUNK, CHUNK)], sem))
        for cp in copies:
            cp.wait()

        def group_body(o, carry2):
            rows = o * LANES + lane
            upar = (uid_v[pl.ds(base + o * LANES, LANES)] & 1) * EMB
            for w in range(HALF):
                rot = (w + lane) & (HALF - 1)
                lo = plsc.load_gather(udat_v, [rows, upar + rot])
                hi = plsc.load_gather(udat_v, [rows, upar + rot + HALF])
                packed = plsc.bitcast(
                    plsc.pack(lo, hi, format=plsc.PackFormat.INTERLEAVED),
                    jnp.int32)
                plsc.store_scatter(ustg_v, [base + rows, rot], packed)
            return carry2

        lax.fori_loop(0, WAVE // LANES, group_body, 0)
        return carry

    lax.fori_loop(0, NWAVE, wave_body, 0)
    pltpu.sync_copy(ustg_v, stage_hbm.at[wid])


@functools.partial(
    pl.kernel,
    out_type=jax.ShapeDtypeStruct((NW, BPW), jnp.float32),
    mesh=_mesh,
    compiler_params=_params,
    scratch_types=[
        pltpu.VMEM((BPW,), jnp.int32),
        pltpu.VMEM((BPW,), jnp.int32),
        pltpu.VMEM((WAVE, 128), jnp.float32),
        pltpu.VMEM((BPW, HALF), jnp.int32),
        pltpu.VMEM((BPW,), jnp.float32),
        pltpu.SemaphoreType.DMA,
    ],
)
def _mf_dot_i(i_id_hbm, itab_hbm, stage_hbm, out_hbm,
              iid_v, ikey_v, idat_v, ustg_v, out_v, sem):
    wid = lax.axis_index("s") * NC + lax.axis_index("c")
    pltpu.sync_copy(i_id_hbm.at[wid], iid_v)
    pltpu.sync_copy(stage_hbm.at[wid], ustg_v)
    for o in range(BPW // LANES):
        sl = pl.ds(o * LANES, LANES)
        ikey_v[sl] = lax.shift_right_logical(iid_v[sl], 1)

    lane = lax.iota(jnp.int32, LANES)

    def wave_body(wv, carry):
        base = wv * WAVE
        copies = []
        for c in range(WAVE // CHUNK):
            sl = pl.ds(base + c * CHUNK, CHUNK)
            copies.append(
                pltpu.async_copy(itab_hbm.at[ikey_v.at[sl]],
                                 idat_v.at[pl.ds(c * CHUNK, CHUNK)], sem))
        for cp in copies:
            cp.wait()

        def group_body(o, carry2):
            rows = o * LANES + lane
            ipar = (iid_v[pl.ds(base + o * LANES, LANES)] & 1) * EMB
            acc = jnp.zeros((LANES,), jnp.float32)
            for w in range(HALF):
                rot = (w + lane) & (HALF - 1)
                packed = plsc.load_gather(ustg_v, [base + rows, rot])
                lo, hi = plsc.unpack(
                    plsc.bitcast(packed, jnp.bfloat16),
                    format=plsc.PackFormat.INTERLEAVED)
                ilo = plsc.load_gather(idat_v, [rows, ipar + rot])
                ihi = plsc.load_gather(idat_v, [rows, ipar + rot + HALF])
                acc = acc + lo.astype(jnp.float32) * ilo
                acc = acc + hi.astype(jnp.float32) * ihi
            out_v[pl.ds(base + o * LANES, LANES)] = acc
            return carry2

        lax.fori_loop(0, WAVE // LANES, group_body, 0)
        return carry

    lax.fori_loop(0, NWAVE, wave_body, 0)
    pltpu.sync_copy(out_v, out_hbm.at[wid])


def kernel(u_id, i_id, user_emb, item_emb):
    u2 = u_id.astype(jnp.int32).reshape(NW, BPW)
    i2 = i_id.astype(jnp.int32).reshape(NW, BPW)
    utab = user_emb.reshape(500000, 128)
    itab = item_emb.reshape(500000, 128)
    stage = _mf_gather_u(u2, utab)
    out = _mf_dot_i(i2, itab, stage)
    return out.reshape(BATCH)


# speedup vs baseline: 1.0023x; 1.0023x over previous
import functools

import jax
import jax.numpy as jnp
from jax import lax
from jax.experimental import pallas as pl
from jax.experimental.pallas import tpu as pltpu
from jax.experimental.pallas import tpu_sc as plsc

BATCH = 16384
EMB = 64
NC = 2
NS = 16
NW = NC * NS
BPW = BATCH // NW
WAVE = 256
NWAVE = BPW // WAVE
CHUNK = 128
LANES = 16

_mesh = plsc.VectorSubcoreMesh(core_axis_name="c", subcore_axis_name="s")


@functools.partial(
    pl.kernel,
    out_type=jax.ShapeDtypeStruct((NW, BPW), jnp.float32),
    mesh=_mesh,
    compiler_params=pltpu.CompilerParams(
        needs_layout_passes=False,
    ),
    scratch_types=[
        pltpu.VMEM((BPW,), jnp.int32),
        pltpu.VMEM((BPW,), jnp.int32),
        pltpu.VMEM((BPW,), jnp.int32),
        pltpu.VMEM((BPW,), jnp.int32),
        pltpu.VMEM((WAVE, 128), jnp.float32),
        pltpu.VMEM((WAVE, 128), jnp.float32),
        pltpu.VMEM((BPW,), jnp.float32),
        pltpu.SemaphoreType.DMA,
    ],
)
def _mf_dot_kernel(u_id_hbm, i_id_hbm, utab_hbm, itab_hbm, out_hbm,
                   uid_v, iid_v, ukey_v, ikey_v, udat_v, idat_v, out_v, sem):
    wid = lax.axis_index("s") * NC + lax.axis_index("c")

    pltpu.sync_copy(u_id_hbm.at[wid], uid_v)
    pltpu.sync_copy(i_id_hbm.at[wid], iid_v)

    for o in range(BPW // LANES):
        sl = pl.ds(o * LANES, LANES)
        ukey_v[sl] = lax.shift_right_logical(uid_v[sl], 1)
        ikey_v[sl] = lax.shift_right_logical(iid_v[sl], 1)

    lane = lax.iota(jnp.int32, LANES)

    def wave_body(w, carry):
        base = w * WAVE
        copies = []
        for c in range(WAVE // CHUNK):
            sl = pl.ds(base + c * CHUNK, CHUNK)
            dst = pl.ds(c * CHUNK, CHUNK)
            copies.append(
                pltpu.async_copy(utab_hbm.at[ukey_v.at[sl]],
                                 udat_v.at[dst], sem))
            copies.append(
                pltpu.async_copy(itab_hbm.at[ikey_v.at[sl]],
                                 idat_v.at[dst], sem))
        for cp in copies:
            cp.wait()

        def group_body(o, carry2):
            rows = o * LANES + lane
            sl = pl.ds(base + o * LANES, LANES)
            upar = (uid_v[sl] & 1) * EMB
            ipar = (iid_v[sl] & 1) * EMB
            acc = jnp.zeros((LANES,), jnp.float32)
            for d in range(EMB):
                rot = (d + lane) & (EMB - 1)
                u = plsc.load_gather(udat_v, [rows, upar + rot])
                v = plsc.load_gather(idat_v, [rows, ipar + rot])
                acc = acc + u * v
            out_v[sl] = acc
            return carry2

        lax.fori_loop(0, WAVE // LANES, group_body, 0)
        return carry

    lax.fori_loop(0, NWAVE, wave_body, 0)

    pltpu.sync_copy(out_v, out_hbm.at[wid])


def kernel(u_id, i_id, user_emb, item_emb):
    u2 = u_id.astype(jnp.int32).reshape(NW, BPW)
    i2 = i_id.astype(jnp.int32).reshape(NW, BPW)
    utab, itab = lax.optimization_barrier(
        (user_emb.reshape(500000, 128), item_emb.reshape(500000, 128)))
    out = _mf_dot_kernel(u2, i2, utab, itab)
    return out.reshape(BATCH)
